# baseline (speedup 1.0000x reference)
import jax
import jax.numpy as jnp
from jax import lax
from jax.experimental import pallas as pl
from jax.experimental.pallas import tpu as pltpu
from jax.experimental.pallas import tpu_sc as plsc

B = 16384
L = 50
D = 64
V = 1000000

NC = 2
NS = 16
NW = NC * NS

VB = 32768
XB = 256
TJ = 2
UNITS = L * TJ


def _transpose_body(in_ref, out_ref):
    a = in_ref[...]
    at = a.T
    h = VB // 2
    out_ref[...] = jnp.concatenate([at[0:h], at[h:VB]], axis=1)


def _gather_body(idx_hbm, table_hbm, out_hbm, idx_v, buf0, buf1, tbuf0,
                 tbuf1, sem_g0, sem_g1, sem_o0, sem_o1):
    wid = lax.axis_index("s") * NC + lax.axis_index("c")
    bufs = (buf0, buf1)
    tbufs = (tbuf0, tbuf1)
    sem_g = (sem_g0, sem_g1)
    sem_o = (sem_o0, sem_o1)

    pltpu.sync_copy(idx_hbm.at[pl.ds(wid * (UNITS * XB), UNITS * XB)], idx_v)

    def fire_gather(u, b):
        pltpu.async_copy(
            table_hbm.at[idx_v.at[pl.ds(u * XB, XB)]], bufs[b], sem_g[b]
        )

    def wait_gather(b):
        pltpu.make_async_copy(
            table_hbm.at[pl.ds(0, XB)], bufs[b], sem_g[b]
        ).wait()

    def fire_out(u, b):
        l = u // TJ
        j = u % TJ
        pltpu.async_copy(
            tbufs[b].at[:, pl.ds(0, XB)],
            out_hbm.at[l, :, pl.ds(wid * (TJ * XB) + j * XB, XB)],
            sem_o[b],
        )

    def wait_out(b):
        pltpu.make_async_copy(
            tbufs[b].at[:, pl.ds(0, XB)], out_hbm.at[0, :, pl.ds(0, XB)],
            sem_o[b],
        ).wait()

    def transpose(b):
        lanes = lax.iota(jnp.int32, 16)
        @pl.loop(0, XB, unroll=64)
        def _r(r):
            rvec = jnp.full((16,), 0, jnp.int32) + r
            for kv in range(D // 16):
                vals = bufs[b][r, pl.ds(kv * 16, 16)]
                plsc.store_scatter(tbufs[b], [lanes + kv * 16, rvec], vals)

    fire_gather(0, 0)

    @pl.loop(0, UNITS, step=2)
    def _pair(i):
        for b in range(2):
            u = i + b

            @pl.when(u + 1 < UNITS)
            def _fire_next():
                fire_gather(u + 1, 1 - b)

            wait_gather(b)

            @pl.when(u >= 2)
            def _w():
                wait_out(b)

            transpose(b)
            fire_out(u, b)

    wait_out(0)
    wait_out(1)


@jax.jit
def kernel(x, W):
    nblk = (V + VB - 1) // VB
    wlin2 = pl.pallas_call(
        _transpose_body,
        grid=(nblk,),
        in_specs=[pl.BlockSpec((D, VB), lambda i: (0, i))],
        out_specs=pl.BlockSpec((VB // 2, 128), lambda i: (i, 0)),
        out_shape=jax.ShapeDtypeStruct((nblk * (VB // 2), 128), jnp.float32),
    )(W.T)
    table = wlin2.reshape(nblk * VB, D)

    xi = x.T.astype(jnp.int32)
    g = (
        (xi & jnp.int32(~(VB - 1)))
        | ((xi & jnp.int32(VB // 2 - 1)) << 1)
        | ((xi >> (VB.bit_length() - 2)) & jnp.int32(1))
    )

    idxp = (
        g.reshape(L, NW, TJ, XB)
        .transpose(1, 0, 2, 3)
        .reshape(NW * UNITS * XB)
    )

    mesh = plsc.VectorSubcoreMesh(core_axis_name="c", subcore_axis_name="s")
    op = pl.kernel(
        _gather_body,
        out_type=jax.ShapeDtypeStruct((L, D, B), jnp.float32),
        mesh=mesh,
        compiler_params=pltpu.CompilerParams(
            use_tc_tiling_on_sc=False, needs_layout_passes=False
        ),
        scratch_types=[
            pltpu.VMEM((UNITS * XB,), jnp.int32),
            pltpu.VMEM((XB, D), jnp.float32),
            pltpu.VMEM((XB, D), jnp.float32),
            pltpu.VMEM((D, XB + 1), jnp.float32),
            pltpu.VMEM((D, XB + 1), jnp.float32),
            pltpu.SemaphoreType.DMA,
            pltpu.SemaphoreType.DMA,
            pltpu.SemaphoreType.DMA,
            pltpu.SemaphoreType.DMA,
        ],
    )(idxp, table)
    return op.transpose(2, 0, 1)


# speedup vs baseline: 1.0142x; 1.0142x over previous
import jax
import jax.numpy as jnp
from jax import lax
from jax.experimental import pallas as pl
from jax.experimental.pallas import tpu as pltpu
from jax.experimental.pallas import tpu_sc as plsc

B = 16384
L = 50
D = 64
V = 1000000

NC = 2
NS = 16
NW = NC * NS

VB = 32768
XB = 256
TJ = 2
UNITS = L * TJ


def _transpose_body(in_ref, out_ref):
    a = in_ref[...]
    at = a.T
    h = VB // 2
    out_ref[...] = jnp.concatenate([at[0:h], at[h:VB]], axis=1)


def _gather_body(idx_hbm, table_hbm, out_hbm, idx_v, buf0, buf1, tbuf0,
                 tbuf1, sem_g0, sem_g1, sem_o0, sem_o1):
    wid = lax.axis_index("s") * NC + lax.axis_index("c")
    bufs = (buf0, buf1)
    tbufs = (tbuf0, tbuf1)
    sem_g = (sem_g0, sem_g1)
    sem_o = (sem_o0, sem_o1)

    pltpu.sync_copy(idx_hbm.at[pl.ds(wid * (UNITS * XB), UNITS * XB)], idx_v)

    def fire_gather(u, b):
        pltpu.async_copy(
            table_hbm.at[idx_v.at[pl.ds(u * XB, XB)]], bufs[b], sem_g[b]
        )

    def wait_gather(b):
        pltpu.make_async_copy(
            table_hbm.at[pl.ds(0, XB)], bufs[b], sem_g[b]
        ).wait()

    def fire_out(u, b):
        l = u // TJ
        j = u % TJ
        pltpu.async_copy(
            tbufs[b].at[:, pl.ds(0, XB)],
            out_hbm.at[l, :, pl.ds(wid * (TJ * XB) + j * XB, XB)],
            sem_o[b],
        )

    def wait_out(b):
        pltpu.make_async_copy(
            tbufs[b].at[:, pl.ds(0, XB)], out_hbm.at[0, :, pl.ds(0, XB)],
            sem_o[b],
        ).wait()

    def transpose(b):
        lanes = lax.iota(jnp.int32, 16)
        @pl.loop(0, XB, unroll=32)
        def _r(r):
            rvec = jnp.full((16,), 0, jnp.int32) + r
            for kv in range(D // 16):
                vals = bufs[b][r, pl.ds(kv * 16, 16)]
                plsc.store_scatter(tbufs[b], [lanes + kv * 16, rvec], vals)

    fire_gather(0, 0)

    @pl.loop(0, UNITS, step=2)
    def _pair(i):
        for b in range(2):
            u = i + b

            @pl.when(u + 1 < UNITS)
            def _fire_next():
                fire_gather(u + 1, 1 - b)

            wait_gather(b)

            @pl.when(u >= 2)
            def _w():
                wait_out(b)

            transpose(b)
            fire_out(u, b)

    wait_out(0)
    wait_out(1)


@jax.jit
def kernel(x, W):
    nblk = (V + VB - 1) // VB
    wlin2 = pl.pallas_call(
        _transpose_body,
        grid=(nblk,),
        in_specs=[pl.BlockSpec((D, VB), lambda i: (0, i))],
        out_specs=pl.BlockSpec((VB // 2, 128), lambda i: (i, 0)),
        out_shape=jax.ShapeDtypeStruct((nblk * (VB // 2), 128), jnp.float32),
    )(W.T)
    table = wlin2.reshape(nblk * VB, D)

    xi = x.T.astype(jnp.int32)
    g = (
        (xi & jnp.int32(~(VB - 1)))
        | ((xi & jnp.int32(VB // 2 - 1)) << 1)
        | ((xi >> (VB.bit_length() - 2)) & jnp.int32(1))
    )

    idxp = (
        g.reshape(L, NW, TJ, XB)
        .transpose(1, 0, 2, 3)
        .reshape(NW * UNITS * XB)
    )

    mesh = plsc.VectorSubcoreMesh(core_axis_name="c", subcore_axis_name="s")
    op = pl.kernel(
        _gather_body,
        out_type=jax.ShapeDtypeStruct((L, D, B), jnp.float32),
        mesh=mesh,
        compiler_params=pltpu.CompilerParams(
            use_tc_tiling_on_sc=False, needs_layout_passes=False
        ),
        scratch_types=[
            pltpu.VMEM((UNITS * XB,), jnp.int32),
            pltpu.VMEM((XB, D), jnp.float32),
            pltpu.VMEM((XB, D), jnp.float32),
            pltpu.VMEM((D, XB + 1), jnp.float32),
            pltpu.VMEM((D, XB + 1), jnp.float32),
            pltpu.SemaphoreType.DMA,
            pltpu.SemaphoreType.DMA,
            pltpu.SemaphoreType.DMA,
            pltpu.SemaphoreType.DMA,
        ],
    )(idxp, table)
    return op.transpose(2, 0, 1)
